# Initial kernel scaffold; baseline (speedup 1.0000x reference)
import jax
import jax.numpy as jnp
from jax.experimental import pallas as pl


def kernel(input, weight):
    raise NotImplementedError("write your pallas kernel here")


# speedup vs baseline: 9.3268x; 9.3268x over previous
import jax
import jax.numpy as jnp
from jax.experimental import pallas as pl
from jax.experimental.pallas import tpu as pltpu
from jax.experimental.pallas import tpu_sc as plsc


def kernel(input, weight):
    B, S = input.shape
    N, D1, D2 = weight.shape
    D = D1 * D2
    num_indices = B * S

    table = weight.reshape(N, D)
    idx = input.reshape(1, num_indices).astype(jnp.int32)

    WINDOW = 128
    assert num_indices % WINDOW == 0

    mesh = plsc.VectorSubcoreMesh(
        core_axis_name="core", subcore_axis_name="subcore"
    )

    @pl.kernel(
        out_type=jax.ShapeDtypeStruct((num_indices, D), table.dtype),
        mesh=mesh,
    )
    def sc_gather(x_hbm, i_hbm, o_hbm):
        def body(i_vmem, o_vmem):
            pltpu.sync_copy(x_hbm.at[i_vmem.at[0]], o_vmem)

        pltpu.emit_pipeline(
            body,
            grid=(num_indices // WINDOW,),
            in_specs=[
                pl.BlockSpec((1, WINDOW), index_map=lambda i: (0, i))
            ],
            out_specs=[
                pl.BlockSpec((WINDOW, D), index_map=lambda i: (i, 0))
            ],
            core_axis_name=("core", "subcore"),
            dimension_semantics=(pltpu.PARALLEL,),
        )(i_hbm, o_hbm)

    out = sc_gather(table, idx)
    return out.reshape(B, S, D1, D2)


# speedup vs baseline: 9.4422x; 1.0124x over previous
import jax
import jax.numpy as jnp
from jax.experimental import pallas as pl
from jax.experimental.pallas import tpu as pltpu
from jax.experimental.pallas import tpu_sc as plsc


def kernel(input, weight):
    B, S = input.shape
    N, D1, D2 = weight.shape
    D = D1 * D2
    num_indices = B * S

    table = weight.reshape(N, D)
    idx = input.reshape(1, num_indices).astype(jnp.int32)

    WINDOW = 256
    assert num_indices % WINDOW == 0

    mesh = plsc.VectorSubcoreMesh(
        core_axis_name="core", subcore_axis_name="subcore"
    )

    @pl.kernel(
        out_type=jax.ShapeDtypeStruct((num_indices, D), table.dtype),
        mesh=mesh,
    )
    def sc_gather(x_hbm, i_hbm, o_hbm):
        def body(i_vmem, o_vmem):
            pltpu.sync_copy(x_hbm.at[i_vmem.at[0]], o_vmem)

        pltpu.emit_pipeline(
            body,
            grid=(num_indices // WINDOW,),
            in_specs=[
                pl.BlockSpec((1, WINDOW), index_map=lambda i: (0, i))
            ],
            out_specs=[
                pl.BlockSpec((WINDOW, D), index_map=lambda i: (i, 0))
            ],
            core_axis_name=("core", "subcore"),
            dimension_semantics=(pltpu.PARALLEL,),
        )(i_hbm, o_hbm)

    out = sc_gather(table, idx)
    return out.reshape(B, S, D1, D2)


# speedup vs baseline: 12.7612x; 1.3515x over previous
import jax
import jax.numpy as jnp
from jax.experimental import pallas as pl
from jax.experimental.pallas import tpu as pltpu
from jax.experimental.pallas import tpu_sc as plsc


def kernel(input, weight):
    B, S = input.shape
    N, D1, D2 = weight.shape
    D = D1 * D2
    num_indices = B * S

    table = weight.reshape(N, D)
    idx = input.reshape(1, num_indices).astype(jnp.int32)

    WINDOW = 256
    assert num_indices % WINDOW == 0

    mesh = plsc.VectorSubcoreMesh(
        core_axis_name="core", subcore_axis_name="subcore"
    )

    @pl.kernel(
        out_type=jax.ShapeDtypeStruct((num_indices, D), weight.dtype),
        mesh=mesh,
    )
    def sc_gather(x_hbm, i_hbm, o_hbm):
        def body(i_vmem, o_vmem):
            pltpu.sync_copy(x_hbm.at[i_vmem.at[0]], o_vmem)

        pltpu.emit_pipeline(
            body,
            grid=(num_indices // WINDOW,),
            in_specs=[
                pl.BlockSpec((1, WINDOW), index_map=lambda i: (0, i))
            ],
            out_specs=[
                pl.BlockSpec((WINDOW, D), index_map=lambda i: (i, 0))
            ],
            core_axis_name=("core", "subcore"),
            dimension_semantics=(pltpu.PARALLEL,),
        )(i_hbm, o_hbm)

    @pl.kernel(
        out_type=jax.ShapeDtypeStruct((num_indices, D1, D2), weight.dtype),
        mesh=mesh,
    )
    def sc_format(g_hbm, o_hbm):
        FW = 32

        def body(g_vmem, o_vmem):
            o_flat = o_vmem.reshape(FW * D1, D2)

            @pl.loop(0, FW)
            def _(r):
                for s in range(D1):
                    o_flat[r * D1 + s, :] = g_vmem[r, pl.ds(s * D2, D2)]

        pltpu.emit_pipeline(
            body,
            grid=(num_indices // FW,),
            in_specs=[
                pl.BlockSpec((FW, D), index_map=lambda i: (i, 0))
            ],
            out_specs=[
                pl.BlockSpec(
                    (FW, D1, D2), index_map=lambda i: (i, 0, 0)
                )
            ],
            core_axis_name=("core", "subcore"),
            dimension_semantics=(pltpu.PARALLEL,),
        )(g_hbm, o_hbm)

    gathered = sc_gather(table, idx)
    out = sc_format(gathered)
    return out.reshape(B, S, D1, D2)
